# Initial kernel scaffold; baseline (speedup 1.0000x reference)
import jax
import jax.numpy as jnp
from jax.experimental import pallas as pl


def kernel(x, Z, scale_param, shift_param):
    raise NotImplementedError("write your pallas kernel here")


# speedup vs baseline: 1086.2618x; 1086.2618x over previous
import jax
import jax.numpy as jnp
from jax import lax
from jax.experimental import pallas as pl
from jax.experimental.pallas import tpu as pltpu
from jax.experimental.pallas import tpu_sc as plsc

NC = 2
NS = 16
NW = NC * NS
L = 16

TABLE_PAD = 128
CHUNK = 16384


def _body(x_hbm, z_hbm, scale_hbm, shift_hbm, out_hbm,
          scale_v, shift_v, xb0, xb1, zb0, zb1, ob0, ob1,
          in0, in1, out0, out1):
    wid = lax.axis_index("s") * NC + lax.axis_index("c")
    n_chunks = x_hbm.shape[0] // (NW * CHUNK)
    base = wid * (n_chunks * CHUNK)

    pltpu.sync_copy(scale_hbm, scale_v)
    pltpu.sync_copy(shift_hbm, shift_v)

    bufs = ((xb0, zb0, ob0, in0, out0), (xb1, zb1, ob1, in1, out1))

    def start_in(c, xb, zb, sem):
        off = base + c * CHUNK
        pltpu.async_copy(x_hbm.at[pl.ds(off, CHUNK)], xb, sem)
        pltpu.async_copy(z_hbm.at[pl.ds(off, CHUNK)], zb, sem)

    def wait_in(c, xb, zb, sem):
        off = base + c * CHUNK
        pltpu.make_async_copy(x_hbm.at[pl.ds(off, CHUNK)], xb, sem).wait()
        pltpu.make_async_copy(z_hbm.at[pl.ds(off, CHUNK)], zb, sem).wait()

    def start_out(c, ob, sem):
        off = base + c * CHUNK
        pltpu.async_copy(ob, out_hbm.at[pl.ds(off, CHUNK)], sem)

    def wait_out(c, ob, sem):
        off = base + c * CHUNK
        pltpu.make_async_copy(ob, out_hbm.at[pl.ds(off, CHUNK)], sem).wait()

    def compute(xb, zb, ob):
        def vec_body(j, carry):
            sl = pl.ds(j * L, L)
            idx = zb[sl]
            s = plsc.load_gather(scale_v, [idx])
            t = plsc.load_gather(shift_v, [idx])
            ob[sl] = s * xb[sl] + t
            return carry
        lax.fori_loop(0, CHUNK // L, vec_body, 0)

    for b in range(2):
        xb, zb, _, sem, _ = bufs[b]
        start_in(b, xb, zb, sem)

    def group_body(g, carry):
        for b in range(2):
            xb, zb, ob, isem, osem = bufs[b]
            c = g * 2 + b
            wait_in(c, xb, zb, isem)

            @pl.when(g > 0)
            def _():
                wait_out(c - 2, ob, osem)

            compute(xb, zb, ob)
            start_out(c, ob, osem)

            @pl.when(c + 2 < n_chunks)
            def _():
                start_in(c + 2, xb, zb, isem)
        return carry

    lax.fori_loop(0, n_chunks // 2, group_body, 0)

    for b in range(2):
        _, _, ob, _, osem = bufs[b]
        wait_out(n_chunks - 2 + b, ob, osem)


def kernel(x, Z, scale_param, shift_param):
    n = x.shape[0]
    assert n % (NW * CHUNK * 2) == 0
    n_rows = scale_param.shape[0]
    scale_pad = jnp.zeros((TABLE_PAD,), jnp.float32).at[:n_rows].set(
        scale_param.astype(jnp.float32))
    shift_pad = jnp.zeros((TABLE_PAD,), jnp.float32).at[:n_rows].set(
        shift_param.astype(jnp.float32))

    mesh = plsc.VectorSubcoreMesh(core_axis_name="c", subcore_axis_name="s")
    run = pl.kernel(
        _body,
        out_type=jax.ShapeDtypeStruct((n,), jnp.float32),
        mesh=mesh,
        scratch_types=[
            pltpu.VMEM((TABLE_PAD,), jnp.float32),
            pltpu.VMEM((TABLE_PAD,), jnp.float32),
            pltpu.VMEM((CHUNK,), jnp.float32),
            pltpu.VMEM((CHUNK,), jnp.float32),
            pltpu.VMEM((CHUNK,), jnp.int32),
            pltpu.VMEM((CHUNK,), jnp.int32),
            pltpu.VMEM((CHUNK,), jnp.float32),
            pltpu.VMEM((CHUNK,), jnp.float32),
            pltpu.SemaphoreType.DMA,
            pltpu.SemaphoreType.DMA,
            pltpu.SemaphoreType.DMA,
            pltpu.SemaphoreType.DMA,
        ],
        compiler_params=pltpu.CompilerParams(needs_layout_passes=False),
    )
    return run(x.astype(jnp.float32), Z, scale_pad, shift_pad)


# speedup vs baseline: 2228.4341x; 2.0515x over previous
import jax
import jax.numpy as jnp
from jax import lax
from jax.experimental import pallas as pl
from jax.experimental.pallas import tpu as pltpu
from jax.experimental.pallas import tpu_sc as plsc

NC = 2
NS = 16
NW = NC * NS
L = 16

TABLE_PAD = 128
CHUNK = 16384


def _body(x_hbm, z_hbm, scale_hbm, shift_hbm, out_hbm,
          scale_v, shift_v, xb0, xb1, zb0, zb1, ob0, ob1,
          in0, in1, out0, out1):
    wid = lax.axis_index("s") * NC + lax.axis_index("c")
    n_chunks = x_hbm.shape[0] // (NW * CHUNK)
    base = wid * (n_chunks * CHUNK)

    pltpu.sync_copy(scale_hbm, scale_v)
    pltpu.sync_copy(shift_hbm, shift_v)

    bufs = ((xb0, zb0, ob0, in0, out0), (xb1, zb1, ob1, in1, out1))

    def start_in(c, xb, zb, sem):
        off = base + c * CHUNK
        pltpu.async_copy(x_hbm.at[pl.ds(off, CHUNK)], xb, sem)
        pltpu.async_copy(z_hbm.at[pl.ds(off, CHUNK)], zb, sem)

    def wait_in(c, xb, zb, sem):
        off = base + c * CHUNK
        pltpu.make_async_copy(x_hbm.at[pl.ds(off, CHUNK)], xb, sem).wait()
        pltpu.make_async_copy(z_hbm.at[pl.ds(off, CHUNK)], zb, sem).wait()

    def start_out(c, ob, sem):
        off = base + c * CHUNK
        pltpu.async_copy(ob, out_hbm.at[pl.ds(off, CHUNK)], sem)

    def wait_out(c, ob, sem):
        off = base + c * CHUNK
        pltpu.make_async_copy(ob, out_hbm.at[pl.ds(off, CHUNK)], sem).wait()

    def compute(xb, zb, ob):
        @plsc.parallel_loop(0, CHUNK, L, unroll=8)
        def vec_body(i):
            sl = pl.ds(i, L)
            idx = zb[sl]
            s = plsc.load_gather(scale_v, [idx])
            t = plsc.load_gather(shift_v, [idx])
            ob[sl] = s * xb[sl] + t

    for b in range(2):
        xb, zb, _, sem, _ = bufs[b]
        start_in(b, xb, zb, sem)

    def group_body(g, carry):
        for b in range(2):
            xb, zb, ob, isem, osem = bufs[b]
            c = g * 2 + b
            wait_in(c, xb, zb, isem)

            @pl.when(g > 0)
            def _():
                wait_out(c - 2, ob, osem)

            compute(xb, zb, ob)
            start_out(c, ob, osem)

            @pl.when(c + 2 < n_chunks)
            def _():
                start_in(c + 2, xb, zb, isem)
        return carry

    lax.fori_loop(0, n_chunks // 2, group_body, 0)

    for b in range(2):
        _, _, ob, _, osem = bufs[b]
        wait_out(n_chunks - 2 + b, ob, osem)


def kernel(x, Z, scale_param, shift_param):
    n = x.shape[0]
    assert n % (NW * CHUNK * 2) == 0
    n_rows = scale_param.shape[0]
    scale_pad = jnp.zeros((TABLE_PAD,), jnp.float32).at[:n_rows].set(
        scale_param.astype(jnp.float32))
    shift_pad = jnp.zeros((TABLE_PAD,), jnp.float32).at[:n_rows].set(
        shift_param.astype(jnp.float32))

    mesh = plsc.VectorSubcoreMesh(core_axis_name="c", subcore_axis_name="s")
    run = pl.kernel(
        _body,
        out_type=jax.ShapeDtypeStruct((n,), jnp.float32),
        mesh=mesh,
        scratch_types=[
            pltpu.VMEM((TABLE_PAD,), jnp.float32),
            pltpu.VMEM((TABLE_PAD,), jnp.float32),
            pltpu.VMEM((CHUNK,), jnp.float32),
            pltpu.VMEM((CHUNK,), jnp.float32),
            pltpu.VMEM((CHUNK,), jnp.int32),
            pltpu.VMEM((CHUNK,), jnp.int32),
            pltpu.VMEM((CHUNK,), jnp.float32),
            pltpu.VMEM((CHUNK,), jnp.float32),
            pltpu.SemaphoreType.DMA,
            pltpu.SemaphoreType.DMA,
            pltpu.SemaphoreType.DMA,
            pltpu.SemaphoreType.DMA,
        ],
        compiler_params=pltpu.CompilerParams(needs_layout_passes=False),
    )
    return run(x.astype(jnp.float32), Z, scale_pad, shift_pad)
